# speedup vs baseline: 3.4198x; 3.4198x over previous
import jax
import jax.numpy as jnp
from jax.experimental import pallas as pl


def _copy_body(t_ref, o_ref):
    o_ref[0] = t_ref[...]


def kernel(x, table):
    B, length, _ = x.shape
    _, D = table.shape
    BLK = 512
    out = pl.pallas_call(
        _copy_body,
        grid=(length // BLK, B),
        in_specs=[pl.BlockSpec((BLK, D), lambda i, b: (i, 0))],
        out_specs=pl.BlockSpec((1, BLK, D), lambda i, b: (b, i, 0)),
        out_shape=jax.ShapeDtypeStruct((B, length, D), table.dtype),
    )(table)
    return out


# speedup vs baseline: 3.6223x; 1.0592x over previous
import functools

import jax
import jax.numpy as jnp
from jax import lax
from jax.experimental import pallas as pl
from jax.experimental.pallas import tpu as pltpu
from jax.experimental.pallas import tpu_sc as plsc

_NC = 2
_NS = 16
_NW = _NC * _NS


def kernel(x, table):
    B, length, _ = x.shape
    V, D = table.shape
    rows_per_w = length // _NW
    CHUNK = 64
    n_chunks = rows_per_w // CHUNK

    mesh = plsc.VectorSubcoreMesh(core_axis_name="c", subcore_axis_name="s")

    @functools.partial(
        pl.kernel,
        out_type=jax.ShapeDtypeStruct((B, length, D), table.dtype),
        mesh=mesh,
        scratch_types=[pltpu.VMEM((CHUNK, D), table.dtype)],
    )
    def sc_copy(table_hbm, out_hbm, buf):
        wid = lax.axis_index("s") * _NC + lax.axis_index("c")
        base = wid * rows_per_w
        for c in range(n_chunks):
            off = base + c * CHUNK
            pltpu.sync_copy(table_hbm.at[pl.ds(off, CHUNK)], buf)
            for b in range(B):
                pltpu.sync_copy(buf, out_hbm.at[b, pl.ds(off, CHUNK)])

    return sc_copy(table)


def _copy_body(t_ref, o_ref):
    o_ref[0] = t_ref[...]


def _kernel_tc(x, table):
    B, length, _ = x.shape
    _, D = table.shape
    BLK = 512
    out = pl.pallas_call(
        _copy_body,
        grid=(length // BLK, B),
        in_specs=[pl.BlockSpec((BLK, D), lambda i, b: (i, 0))],
        out_specs=pl.BlockSpec((1, BLK, D), lambda i, b: (b, i, 0)),
        out_shape=jax.ShapeDtypeStruct((B, length, D), table.dtype),
    )(table)
    return out
